# Initial kernel scaffold; baseline (speedup 1.0000x reference)
import jax
import jax.numpy as jnp
from jax.experimental import pallas as pl


def kernel(z, pos, batch, edge_index, edge_attr, emb, Wf1, bf1, Wf2, bf2, Wl1, Wl2, bl2, Wr1, br1, Wr2, br2):
    raise NotImplementedError("write your pallas kernel here")


# speedup vs baseline: 5.4595x; 5.4595x over previous
import functools

import jax
import jax.numpy as jnp
from jax import lax
from jax.experimental import pallas as pl
from jax.experimental.pallas import tpu as pltpu
from jax.experimental.pallas import tpu_sc as plsc

N = 10000
E = 320000
H = 128
NG = 46
NB = 4
NI = 3
NGRAPH = 32
CUT = 10.0
VOCAB = 100

NC = 2
NS = 16
NW = NC * NS
EPW = E // NW
CE = 40
NCHUNK = EPW // CE
NP = 10240
NPT = NP // NS

F32 = jnp.float32


def ssp(x):
    return jax.nn.softplus(x) - jnp.log(2.0)


def _sc_dd_body(px_h, py_h, pz_h, row_h, col_h, dd_h,
                px_v, py_v, pz_v, row_v, col_v, dd_v):
    wid = lax.axis_index("s") * NC + lax.axis_index("c")
    base = wid * EPW
    pltpu.sync_copy(px_h, px_v)
    pltpu.sync_copy(py_h, py_v)
    pltpu.sync_copy(pz_h, pz_v)
    pltpu.sync_copy(row_h.at[pl.ds(base, EPW)], row_v)
    pltpu.sync_copy(col_h.at[pl.ds(base, EPW)], col_v)

    def step(j, c):
        sl = pl.ds(j * 16, 16)
        r16 = row_v[sl]
        c16 = col_v[sl]
        dx = plsc.load_gather(px_v, [r16]) - plsc.load_gather(px_v, [c16])
        dy = plsc.load_gather(py_v, [r16]) - plsc.load_gather(py_v, [c16])
        dz = plsc.load_gather(pz_v, [r16]) - plsc.load_gather(pz_v, [c16])
        dd_v[sl] = dx * dx + dy * dy + dz * dz
        return c

    lax.fori_loop(0, EPW // 16, step, 0)
    pltpu.sync_copy(dd_v, dd_h.at[pl.ds(base, EPW)])


def _sc_dd(px, py, pz, row, col):
    mesh = plsc.VectorSubcoreMesh(core_axis_name="c", subcore_axis_name="s")
    fn = pl.kernel(
        _sc_dd_body,
        out_type=jax.ShapeDtypeStruct((E,), F32),
        mesh=mesh,
        scratch_types=[
            pltpu.VMEM((N,), F32),
            pltpu.VMEM((N,), F32),
            pltpu.VMEM((N,), F32),
            pltpu.VMEM((EPW,), jnp.int32),
            pltpu.VMEM((EPW,), jnp.int32),
            pltpu.VMEM((EPW,), F32),
        ],
        compiler_params=pltpu.CompilerParams(needs_layout_passes=False),
    )
    return fn(px, py, pz, row, col)


NBUF = 2
NOUTER = NCHUNK // NBUF


def _sc_msg_body(hw_h, we_h, row2_h, col_h, zero_h, out_h,
                 col_v, aggs, gsem, wsem, rsem, *bufs):
    gath_b = bufs[0:NBUF]
    we_b = bufs[NBUF:2 * NBUF]
    row_b = bufs[2 * NBUF:3 * NBUF]
    cid = lax.axis_index("c")
    sid = lax.axis_index("s")
    wid = sid * NC + cid
    nsl = pl.ds(sid * NPT, NPT)
    pltpu.sync_copy(zero_h.at[nsl, :], aggs.at[nsl, :])
    pltpu.sync_copy(col_h.at[pl.ds(wid * EPW, EPW)], col_v)
    plsc.subcore_barrier()

    def fire(j, b):
        ebase = wid * EPW + j * CE
        pltpu.make_async_copy(hw_h.at[col_v.at[pl.ds(j * CE, CE)]],
                              gath_b[b], gsem.at[b]).start()
        pltpu.make_async_copy(we_h.at[pl.ds(ebase, CE), :], we_b[b],
                              wsem.at[b]).start()
        pltpu.make_async_copy(row2_h.at[wid, j], row_b[b],
                              rsem.at[b]).start()

    def wait_data(j, b):
        pltpu.make_async_copy(hw_h.at[col_v.at[pl.ds(j * CE, CE)]],
                              gath_b[b], gsem.at[b]).wait()
        pltpu.make_async_copy(we_h.at[pl.ds(wid * EPW + j * CE, CE), :],
                              we_b[b], wsem.at[b]).wait()

    def wait_row(j, b):
        pltpu.make_async_copy(row2_h.at[wid, j], row_b[b],
                              rsem.at[b]).wait()

    for b in range(NBUF):
        fire(b, b)

    def outer(j2, c):
        for b in range(NBUF):
            j = j2 * NBUF + b
            wait_data(j, b)
            gb = gath_b[b]
            wb = we_b[b]

            def rowmul(r, c2):
                for k in range(H // 16):
                    sl = pl.ds(k * 16, 16)
                    gb[r, sl] = gb[r, sl] * wb[r, sl]
                return c2

            lax.fori_loop(0, CE, rowmul, 0)
            wait_row(j, b)
            pltpu.sync_copy(gb, aggs.at[row_b[b].at[0]], add=True)

            @pl.when(j2 < NOUTER - 1)
            def _():
                fire(j + NBUF, b)
        return c

    lax.fori_loop(0, NOUTER, outer, 0)
    plsc.subcore_barrier()
    pltpu.sync_copy(aggs.at[nsl, :], out_h.at[cid, nsl, :])


def _sc_msg(hw, we, row2, col, zeros):
    mesh = plsc.VectorSubcoreMesh(core_axis_name="c", subcore_axis_name="s")
    fn = pl.kernel(
        _sc_msg_body,
        out_type=jax.ShapeDtypeStruct((NC, NP, H), F32),
        mesh=mesh,
        scratch_types=(
            [
                pltpu.VMEM((EPW,), jnp.int32),
                pltpu.VMEM_SHARED((NP, H), F32),
                pltpu.SemaphoreType.DMA((NBUF,)),
                pltpu.SemaphoreType.DMA((NBUF,)),
                pltpu.SemaphoreType.DMA((NBUF,)),
            ]
            + [pltpu.VMEM((CE, H), F32) for _ in range(2 * NBUF)]
            + [pltpu.VMEM((1, CE), jnp.int32) for _ in range(NBUF)]
        ),
    )
    return fn(hw, we, row2, col, zeros)


TN = 1000


def _tc_embed_body(z_ref, emb_ref, wl1_ref, h_ref, hw_ref):
    zt = z_ref[0, 0, :]
    oh = (zt[:, None] == lax.broadcasted_iota(jnp.int32, (TN, VOCAB), 1)
          ).astype(F32)
    h = jnp.dot(oh, emb_ref[...], preferred_element_type=F32)
    h_ref[...] = h
    hw_ref[...] = jnp.dot(h, wl1_ref[...], preferred_element_type=F32)


def _tc_embed(z, emb, wl10):
    z3 = z.reshape(N // TN, 1, TN)
    return pl.pallas_call(
        _tc_embed_body,
        grid=(N // TN,),
        in_specs=[
            pl.BlockSpec((1, 1, TN), lambda i: (i, 0, 0)),
            pl.BlockSpec((VOCAB, H), lambda i: (0, 0)),
            pl.BlockSpec((H, H), lambda i: (0, 0)),
        ],
        out_specs=[
            pl.BlockSpec((TN, H), lambda i: (i, 0)),
            pl.BlockSpec((TN, H), lambda i: (i, 0)),
        ],
        out_shape=[
            jax.ShapeDtypeStruct((N, H), F32),
            jax.ShapeDtypeStruct((NP, H), F32),
        ],
    )(z3, emb, wl10)


TE = 2000


def _tc_filter_body(dd_ref, eat_ref, wf1a_ref, wf1g_ref, bf1_ref,
                    wf2_ref, bf2_ref, o_ref):
    dd = dd_ref[0, 0, :]
    ew = jnp.sqrt(dd + 1e-12)
    step = CUT / (NG - 1)
    coeff = -0.5 / (step * step)
    offs = lax.broadcasted_iota(jnp.int32, (TE, NG), 1).astype(F32) * step
    diff = ew[:, None] - offs
    g = jnp.exp(coeff * diff * diff)
    C = 0.5 * (jnp.cos(ew * (jnp.pi / CUT)) + 1.0)
    ea = eat_ref[0]
    t = (jnp.dot(ea, wf1a_ref[...], preferred_element_type=F32)
         + jnp.dot(g, wf1g_ref[...], preferred_element_type=F32)
         + bf1_ref[...])
    t = ssp(t)
    o_ref[...] = (jnp.dot(t.astype(jnp.bfloat16),
                          wf2_ref[...].astype(jnp.bfloat16),
                          preferred_element_type=F32)
                  + bf2_ref[...]) * C[:, None]


def _tc_filter(dd3, ea3, wf1a, wf1g, bf1, wf2, bf2):
    return pl.pallas_call(
        _tc_filter_body,
        grid=(E // TE,),
        in_specs=[
            pl.BlockSpec((1, 1, TE), lambda i: (i, 0, 0)),
            pl.BlockSpec((1, TE, NB), lambda i: (i, 0, 0)),
            pl.BlockSpec((NB, H), lambda i: (0, 0)),
            pl.BlockSpec((NG, H), lambda i: (0, 0)),
            pl.BlockSpec((1, H), lambda i: (0, 0)),
            pl.BlockSpec((H, H), lambda i: (0, 0)),
            pl.BlockSpec((1, H), lambda i: (0, 0)),
        ],
        out_specs=pl.BlockSpec((TE, H), lambda i: (i, 0)),
        out_shape=jax.ShapeDtypeStruct((E, H), F32),
    )(dd3, ea3, wf1a, wf1g, bf1, wf2, bf2)


def _tc_update_body(agg_ref, h_ref, wl2_ref, bl2_ref, wl1n_ref,
                    hn_ref, hwn_ref):
    agg = agg_ref[0] + agg_ref[1]
    hn = h_ref[...] + jnp.dot(ssp(agg), wl2_ref[...],
                              preferred_element_type=F32) + bl2_ref[...]
    hn_ref[...] = hn
    hwn_ref[...] = jnp.dot(hn, wl1n_ref[...], preferred_element_type=F32)


def _tc_update(agg2, h, wl2, bl2, wl1n):
    return pl.pallas_call(
        _tc_update_body,
        grid=(N // TN,),
        in_specs=[
            pl.BlockSpec((NC, TN, H), lambda i: (0, i, 0)),
            pl.BlockSpec((TN, H), lambda i: (i, 0)),
            pl.BlockSpec((H, H), lambda i: (0, 0)),
            pl.BlockSpec((1, H), lambda i: (0, 0)),
            pl.BlockSpec((H, H), lambda i: (0, 0)),
        ],
        out_specs=[
            pl.BlockSpec((TN, H), lambda i: (i, 0)),
            pl.BlockSpec((TN, H), lambda i: (i, 0)),
        ],
        out_shape=[
            jax.ShapeDtypeStruct((N, H), F32),
            jax.ShapeDtypeStruct((NP, H), F32),
        ],
    )(agg2, h, wl2, bl2.reshape(1, H), wl1n)


def _tc_final_body(agg_ref, h_ref, wl2_ref, bl2_ref, wr1_ref, br1_ref,
                   wr2_ref, br2_ref, batch_ref, out_ref):
    agg = agg_ref[0] + agg_ref[1]
    h3 = h_ref[...] + jnp.dot(ssp(agg), wl2_ref[...],
                              preferred_element_type=F32) + bl2_ref[...]
    u = ssp(jnp.dot(h3, wr1_ref[...], preferred_element_type=F32)
            + br1_ref[...])
    hq = jnp.dot(u, wr2_ref[...], preferred_element_type=F32) + br2_ref[...]
    bt = batch_ref[0, 0, :]
    oh = (bt[:, None] == lax.broadcasted_iota(jnp.int32, (TN, NGRAPH), 1)
          ).astype(F32)
    contrib = lax.dot_general(oh, hq, (((0,), (0,)), ((), ())),
                              preferred_element_type=F32)

    @pl.when(pl.program_id(0) == 0)
    def _():
        out_ref[...] = jnp.zeros_like(out_ref)

    out_ref[...] += contrib


def _tc_final(agg2, h, wl2, bl2, Wr1, br1, Wr2, br2, batch):
    wr2p = jnp.pad(Wr2, ((0, 0), (0, H - Wr2.shape[1])))
    b3 = batch.reshape(N // TN, 1, TN)
    out128 = pl.pallas_call(
        _tc_final_body,
        grid=(N // TN,),
        in_specs=[
            pl.BlockSpec((NC, TN, H), lambda i: (0, i, 0)),
            pl.BlockSpec((TN, H), lambda i: (i, 0)),
            pl.BlockSpec((H, H), lambda i: (0, 0)),
            pl.BlockSpec((1, H), lambda i: (0, 0)),
            pl.BlockSpec((H, H // 2), lambda i: (0, 0)),
            pl.BlockSpec((1, H // 2), lambda i: (0, 0)),
            pl.BlockSpec((H // 2, H), lambda i: (0, 0)),
            pl.BlockSpec((1, 1), lambda i: (0, 0)),
            pl.BlockSpec((1, 1, TN), lambda i: (i, 0, 0)),
        ],
        out_specs=pl.BlockSpec((NGRAPH, H), lambda i: (0, 0)),
        out_shape=jax.ShapeDtypeStruct((NGRAPH, H), F32),
    )(agg2, h, wl2, bl2.reshape(1, H), Wr1, br1.reshape(1, H // 2),
      wr2p, br2.reshape(1, 1), b3)
    return out128[:, :1]


def kernel(z, pos, batch, edge_index, edge_attr, emb, Wf1, bf1, Wf2, bf2,
           Wl1, Wl2, bl2, Wr1, br1, Wr2, br2):
    row = edge_index[0].astype(jnp.int32)
    col = edge_index[1].astype(jnp.int32)
    px = pos[:, 0]
    py = pos[:, 1]
    pz = pos[:, 2]

    row2 = row.reshape(NW, NCHUNK, 1, CE)

    dd = _sc_dd(px, py, pz, row, col)
    dd3 = dd.reshape(E // TE, 1, TE)
    ea3 = edge_attr.reshape(E // TE, TE, NB)
    wf1a = Wf1[:, :NB, :]
    wf1g = Wf1[:, NB:, :]

    def filt(i):
        return _tc_filter(dd3, ea3, wf1a[i], wf1g[i], bf1[i].reshape(1, H),
                          Wf2[i], bf2[i].reshape(1, H))

    h, hw = _tc_embed(z.astype(jnp.int32), emb, Wl1[0])
    zeros = jnp.zeros((NP, H), F32)
    we_cur = filt(0)
    for i in range(NI - 1):
        agg2 = _sc_msg(hw, we_cur, row2, col, zeros)
        we_cur = filt(i + 1)
        h, hw = _tc_update(agg2, h, Wl2[i], bl2[i], Wl1[i + 1])
    agg2 = _sc_msg(hw, we_cur, row2, col, zeros)
    return _tc_final(agg2, h, Wl2[NI - 1], bl2[NI - 1],
                     Wr1, br1, Wr2, br2, batch.astype(jnp.int32))


# speedup vs baseline: 5.7682x; 1.0565x over previous
import functools

import jax
import jax.numpy as jnp
from jax import lax
from jax.experimental import pallas as pl
from jax.experimental.pallas import tpu as pltpu
from jax.experimental.pallas import tpu_sc as plsc

N = 10000
E = 320000
H = 128
NG = 46
NB = 4
NI = 3
NGRAPH = 32
CUT = 10.0
VOCAB = 100

NC = 2
NS = 16
NW = NC * NS
EPW = E // NW
CE = 40
NCHUNK = EPW // CE
NP = 10240
NPT = NP // NS

F32 = jnp.float32


def ssp(x):
    return jax.nn.softplus(x) - jnp.log(2.0)


def _sc_dd_body(px_h, py_h, pz_h, row_h, col_h, dd_h,
                px_v, py_v, pz_v, row_v, col_v, dd_v):
    wid = lax.axis_index("s") * NC + lax.axis_index("c")
    base = wid * EPW
    pltpu.sync_copy(px_h, px_v)
    pltpu.sync_copy(py_h, py_v)
    pltpu.sync_copy(pz_h, pz_v)
    pltpu.sync_copy(row_h.at[pl.ds(base, EPW)], row_v)
    pltpu.sync_copy(col_h.at[pl.ds(base, EPW)], col_v)

    def step(j, c):
        sl = pl.ds(j * 16, 16)
        r16 = row_v[sl]
        c16 = col_v[sl]
        dx = plsc.load_gather(px_v, [r16]) - plsc.load_gather(px_v, [c16])
        dy = plsc.load_gather(py_v, [r16]) - plsc.load_gather(py_v, [c16])
        dz = plsc.load_gather(pz_v, [r16]) - plsc.load_gather(pz_v, [c16])
        dd_v[sl] = dx * dx + dy * dy + dz * dz
        return c

    lax.fori_loop(0, EPW // 16, step, 0)
    pltpu.sync_copy(dd_v, dd_h.at[pl.ds(base, EPW)])


def _sc_dd(px, py, pz, row, col):
    mesh = plsc.VectorSubcoreMesh(core_axis_name="c", subcore_axis_name="s")
    fn = pl.kernel(
        _sc_dd_body,
        out_type=jax.ShapeDtypeStruct((E,), F32),
        mesh=mesh,
        scratch_types=[
            pltpu.VMEM((N,), F32),
            pltpu.VMEM((N,), F32),
            pltpu.VMEM((N,), F32),
            pltpu.VMEM((EPW,), jnp.int32),
            pltpu.VMEM((EPW,), jnp.int32),
            pltpu.VMEM((EPW,), F32),
        ],
        compiler_params=pltpu.CompilerParams(needs_layout_passes=False),
    )
    return fn(px, py, pz, row, col)


NBUF = 2
NOUTER = NCHUNK // NBUF


def _sc_msg_body(hw_h, we_h, row2_h, col_h, zero_h, out_h,
                 col_v, aggs, gsem, wsem, rsem, *bufs):
    gath_b = bufs[0:NBUF]
    we_b = bufs[NBUF:2 * NBUF]
    row_b = bufs[2 * NBUF:3 * NBUF]
    cid = lax.axis_index("c")
    sid = lax.axis_index("s")
    wid = sid * NC + cid
    nsl = pl.ds(sid * NPT, NPT)
    pltpu.sync_copy(zero_h.at[nsl, :], aggs.at[nsl, :])
    pltpu.sync_copy(col_h.at[pl.ds(wid * EPW, EPW)], col_v)
    plsc.subcore_barrier()

    def fire(j, b):
        ebase = wid * EPW + j * CE
        pltpu.make_async_copy(hw_h.at[col_v.at[pl.ds(j * CE, CE)]],
                              gath_b[b], gsem.at[b]).start()
        pltpu.make_async_copy(we_h.at[pl.ds(ebase, CE), :], we_b[b],
                              wsem.at[b]).start()
        pltpu.make_async_copy(row2_h.at[wid, j], row_b[b],
                              rsem.at[b]).start()

    def wait_data(j, b):
        pltpu.make_async_copy(hw_h.at[col_v.at[pl.ds(j * CE, CE)]],
                              gath_b[b], gsem.at[b]).wait()
        pltpu.make_async_copy(we_h.at[pl.ds(wid * EPW + j * CE, CE), :],
                              we_b[b], wsem.at[b]).wait()

    def wait_row(j, b):
        pltpu.make_async_copy(row2_h.at[wid, j], row_b[b],
                              rsem.at[b]).wait()

    for b in range(NBUF):
        fire(b, b)

    def outer(j2, c):
        for b in range(NBUF):
            j = j2 * NBUF + b
            wait_data(j, b)
            gb = gath_b[b]
            wb = we_b[b]

            def rowmul(r, c2):
                for k in range(H // 16):
                    sl = pl.ds(k * 16, 16)
                    gb[r, sl] = gb[r, sl] * wb[r, sl]
                return c2

            lax.fori_loop(0, CE, rowmul, 0)
            wait_row(j, b)
            pltpu.sync_copy(gb, aggs.at[row_b[b].at[0]], add=True)

            @pl.when(j2 < NOUTER - 1)
            def _():
                fire(j + NBUF, b)
        return c

    lax.fori_loop(0, NOUTER, outer, 0)
    plsc.subcore_barrier()
    pltpu.sync_copy(aggs.at[nsl, :], out_h.at[cid, nsl, :])


def _sc_msg(hw, we, row2, col, zeros):
    mesh = plsc.VectorSubcoreMesh(core_axis_name="c", subcore_axis_name="s")
    fn = pl.kernel(
        _sc_msg_body,
        out_type=jax.ShapeDtypeStruct((NC, NP, H), F32),
        mesh=mesh,
        scratch_types=(
            [
                pltpu.VMEM((EPW,), jnp.int32),
                pltpu.VMEM_SHARED((NP, H), F32),
                pltpu.SemaphoreType.DMA((NBUF,)),
                pltpu.SemaphoreType.DMA((NBUF,)),
                pltpu.SemaphoreType.DMA((NBUF,)),
            ]
            + [pltpu.VMEM((CE, H), F32) for _ in range(2 * NBUF)]
            + [pltpu.VMEM((1, CE), jnp.int32) for _ in range(NBUF)]
        ),
    )
    return fn(hw, we, row2, col, zeros)


TN = 1000


def _tc_embed_body(z_ref, emb_ref, wl1_ref, h_ref, hw_ref):
    zt = z_ref[0, 0, :]
    oh = (zt[:, None] == lax.broadcasted_iota(jnp.int32, (TN, VOCAB), 1)
          ).astype(F32)
    h = jnp.dot(oh, emb_ref[...], preferred_element_type=F32)
    h_ref[...] = h
    hw_ref[...] = jnp.dot(h, wl1_ref[...], preferred_element_type=F32)


def _tc_embed(z, emb, wl10):
    z3 = z.reshape(N // TN, 1, TN)
    return pl.pallas_call(
        _tc_embed_body,
        grid=(N // TN,),
        in_specs=[
            pl.BlockSpec((1, 1, TN), lambda i: (i, 0, 0)),
            pl.BlockSpec((VOCAB, H), lambda i: (0, 0)),
            pl.BlockSpec((H, H), lambda i: (0, 0)),
        ],
        out_specs=[
            pl.BlockSpec((TN, H), lambda i: (i, 0)),
            pl.BlockSpec((TN, H), lambda i: (i, 0)),
        ],
        out_shape=[
            jax.ShapeDtypeStruct((N, H), F32),
            jax.ShapeDtypeStruct((NP, H), F32),
        ],
    )(z3, emb, wl10)


TE = 2000


def _tc_filter_body(dd_ref, eat_ref, wf1a_ref, wf1g_ref, bf1_ref,
                    wf2_ref, bf2_ref, o_ref):
    dd = dd_ref[0, 0, :]
    ew = jnp.sqrt(dd + 1e-12)
    step = CUT / (NG - 1)
    coeff = -0.5 / (step * step)
    offs = lax.broadcasted_iota(jnp.int32, (TE, NG), 1).astype(F32) * step
    diff = ew[:, None] - offs
    g = jnp.exp(coeff * diff * diff)
    C = 0.5 * (jnp.cos(ew * (jnp.pi / CUT)) + 1.0)
    ea = eat_ref[0]
    t = (jnp.dot(ea, wf1a_ref[...], preferred_element_type=F32)
         + jnp.dot(g, wf1g_ref[...], preferred_element_type=F32)
         + bf1_ref[...])
    t = ssp(t)
    o_ref[...] = (jnp.dot(t, wf2_ref[...], preferred_element_type=F32)
                  + bf2_ref[...]) * C[:, None]


def _tc_filter(dd3, ea3, wf1a, wf1g, bf1, wf2, bf2):
    return pl.pallas_call(
        _tc_filter_body,
        grid=(E // TE,),
        in_specs=[
            pl.BlockSpec((1, 1, TE), lambda i: (i, 0, 0)),
            pl.BlockSpec((1, TE, NB), lambda i: (i, 0, 0)),
            pl.BlockSpec((NB, H), lambda i: (0, 0)),
            pl.BlockSpec((NG, H), lambda i: (0, 0)),
            pl.BlockSpec((1, H), lambda i: (0, 0)),
            pl.BlockSpec((H, H), lambda i: (0, 0)),
            pl.BlockSpec((1, H), lambda i: (0, 0)),
        ],
        out_specs=pl.BlockSpec((TE, H), lambda i: (i, 0)),
        out_shape=jax.ShapeDtypeStruct((E, H), F32),
    )(dd3, ea3, wf1a, wf1g, bf1, wf2, bf2)


def _tc_update_body(agg_ref, h_ref, wl2_ref, bl2_ref, wl1n_ref,
                    hn_ref, hwn_ref):
    agg = agg_ref[0] + agg_ref[1]
    hn = h_ref[...] + jnp.dot(ssp(agg), wl2_ref[...],
                              preferred_element_type=F32) + bl2_ref[...]
    hn_ref[...] = hn
    hwn_ref[...] = jnp.dot(hn, wl1n_ref[...], preferred_element_type=F32)


def _tc_update(agg2, h, wl2, bl2, wl1n):
    return pl.pallas_call(
        _tc_update_body,
        grid=(N // TN,),
        in_specs=[
            pl.BlockSpec((NC, TN, H), lambda i: (0, i, 0)),
            pl.BlockSpec((TN, H), lambda i: (i, 0)),
            pl.BlockSpec((H, H), lambda i: (0, 0)),
            pl.BlockSpec((1, H), lambda i: (0, 0)),
            pl.BlockSpec((H, H), lambda i: (0, 0)),
        ],
        out_specs=[
            pl.BlockSpec((TN, H), lambda i: (i, 0)),
            pl.BlockSpec((TN, H), lambda i: (i, 0)),
        ],
        out_shape=[
            jax.ShapeDtypeStruct((N, H), F32),
            jax.ShapeDtypeStruct((NP, H), F32),
        ],
    )(agg2, h, wl2, bl2.reshape(1, H), wl1n)


def _tc_final_body(agg_ref, h_ref, wl2_ref, bl2_ref, wr1_ref, br1_ref,
                   wr2_ref, br2_ref, batch_ref, out_ref):
    agg = agg_ref[0] + agg_ref[1]
    h3 = h_ref[...] + jnp.dot(ssp(agg), wl2_ref[...],
                              preferred_element_type=F32) + bl2_ref[...]
    u = ssp(jnp.dot(h3, wr1_ref[...], preferred_element_type=F32)
            + br1_ref[...])
    hq = jnp.dot(u, wr2_ref[...], preferred_element_type=F32) + br2_ref[...]
    bt = batch_ref[0, 0, :]
    oh = (bt[:, None] == lax.broadcasted_iota(jnp.int32, (TN, NGRAPH), 1)
          ).astype(F32)
    contrib = lax.dot_general(oh, hq, (((0,), (0,)), ((), ())),
                              preferred_element_type=F32)

    @pl.when(pl.program_id(0) == 0)
    def _():
        out_ref[...] = jnp.zeros_like(out_ref)

    out_ref[...] += contrib


def _tc_final(agg2, h, wl2, bl2, Wr1, br1, Wr2, br2, batch):
    wr2p = jnp.pad(Wr2, ((0, 0), (0, H - Wr2.shape[1])))
    b3 = batch.reshape(N // TN, 1, TN)
    out128 = pl.pallas_call(
        _tc_final_body,
        grid=(N // TN,),
        in_specs=[
            pl.BlockSpec((NC, TN, H), lambda i: (0, i, 0)),
            pl.BlockSpec((TN, H), lambda i: (i, 0)),
            pl.BlockSpec((H, H), lambda i: (0, 0)),
            pl.BlockSpec((1, H), lambda i: (0, 0)),
            pl.BlockSpec((H, H // 2), lambda i: (0, 0)),
            pl.BlockSpec((1, H // 2), lambda i: (0, 0)),
            pl.BlockSpec((H // 2, H), lambda i: (0, 0)),
            pl.BlockSpec((1, 1), lambda i: (0, 0)),
            pl.BlockSpec((1, 1, TN), lambda i: (i, 0, 0)),
        ],
        out_specs=pl.BlockSpec((NGRAPH, H), lambda i: (0, 0)),
        out_shape=jax.ShapeDtypeStruct((NGRAPH, H), F32),
    )(agg2, h, wl2, bl2.reshape(1, H), Wr1, br1.reshape(1, H // 2),
      wr2p, br2.reshape(1, 1), b3)
    return out128[:, :1]


def kernel(z, pos, batch, edge_index, edge_attr, emb, Wf1, bf1, Wf2, bf2,
           Wl1, Wl2, bl2, Wr1, br1, Wr2, br2):
    row = edge_index[0].astype(jnp.int32)
    col = edge_index[1].astype(jnp.int32)
    px = pos[:, 0]
    py = pos[:, 1]
    pz = pos[:, 2]

    row2 = row.reshape(NW, NCHUNK, 1, CE)

    dd = _sc_dd(px, py, pz, row, col)
    dd3 = dd.reshape(E // TE, 1, TE)
    ea3 = edge_attr.reshape(E // TE, TE, NB)
    wf1a = Wf1[:, :NB, :]
    wf1g = Wf1[:, NB:, :]

    def filt(i):
        return _tc_filter(dd3, ea3, wf1a[i], wf1g[i], bf1[i].reshape(1, H),
                          Wf2[i], bf2[i].reshape(1, H))

    h, hw = _tc_embed(z.astype(jnp.int32), emb, Wl1[0])
    zeros = jnp.zeros((NP, H), F32)
    we_cur = filt(0)
    for i in range(NI - 1):
        agg2 = _sc_msg(hw, we_cur, row2, col, zeros)
        we_cur = filt(i + 1)
        h, hw = _tc_update(agg2, h, Wl2[i], bl2[i], Wl1[i + 1])
    agg2 = _sc_msg(hw, we_cur, row2, col, zeros)
    return _tc_final(agg2, h, Wl2[NI - 1], bl2[NI - 1],
                     Wr1, br1, Wr2, br2, batch.astype(jnp.int32))


# speedup vs baseline: 5.9448x; 1.0306x over previous
import functools

import jax
import jax.numpy as jnp
from jax import lax
from jax.experimental import pallas as pl
from jax.experimental.pallas import tpu as pltpu
from jax.experimental.pallas import tpu_sc as plsc

N = 10000
E = 320000
H = 128
NG = 46
NB = 4
NI = 3
NGRAPH = 32
CUT = 10.0
VOCAB = 100

NC = 2
NS = 16
NW = NC * NS
EPW = E // NW
CE = 40
NCHUNK = EPW // CE
NP = 10240
NPT = NP // NS

F32 = jnp.float32


def ssp(x):
    return jax.nn.softplus(x) - jnp.log(2.0)


def _sc_dd_body(px_h, py_h, pz_h, row_h, col_h, dd_h,
                px_v, py_v, pz_v, row_v, col_v, dd_v):
    wid = lax.axis_index("s") * NC + lax.axis_index("c")
    base = wid * EPW
    pltpu.sync_copy(px_h, px_v)
    pltpu.sync_copy(py_h, py_v)
    pltpu.sync_copy(pz_h, pz_v)
    pltpu.sync_copy(row_h.at[pl.ds(base, EPW)], row_v)
    pltpu.sync_copy(col_h.at[pl.ds(base, EPW)], col_v)

    def step(j, c):
        sl = pl.ds(j * 16, 16)
        r16 = row_v[sl]
        c16 = col_v[sl]
        dx = plsc.load_gather(px_v, [r16]) - plsc.load_gather(px_v, [c16])
        dy = plsc.load_gather(py_v, [r16]) - plsc.load_gather(py_v, [c16])
        dz = plsc.load_gather(pz_v, [r16]) - plsc.load_gather(pz_v, [c16])
        dd_v[sl] = dx * dx + dy * dy + dz * dz
        return c

    lax.fori_loop(0, EPW // 16, step, 0)
    pltpu.sync_copy(dd_v, dd_h.at[pl.ds(base, EPW)])


def _sc_dd(px, py, pz, row, col):
    mesh = plsc.VectorSubcoreMesh(core_axis_name="c", subcore_axis_name="s")
    fn = pl.kernel(
        _sc_dd_body,
        out_type=jax.ShapeDtypeStruct((E,), F32),
        mesh=mesh,
        scratch_types=[
            pltpu.VMEM((N,), F32),
            pltpu.VMEM((N,), F32),
            pltpu.VMEM((N,), F32),
            pltpu.VMEM((EPW,), jnp.int32),
            pltpu.VMEM((EPW,), jnp.int32),
            pltpu.VMEM((EPW,), F32),
        ],
        compiler_params=pltpu.CompilerParams(needs_layout_passes=False),
    )
    return fn(px, py, pz, row, col)


NBUF = 3
NGROUP = (NCHUNK - 1) // NBUF
assert NGROUP * NBUF == NCHUNK - 1


def _sc_msg_body(hw_h, we_h, row2_h, col_h, zero_h, out_h,
                 col_v, aggs, gsem, wsem, rsem, ssem, *bufs):
    gath_b = bufs[0:NBUF]
    we_b = bufs[NBUF:2 * NBUF]
    row_b = bufs[2 * NBUF:3 * NBUF]
    cid = lax.axis_index("c")
    sid = lax.axis_index("s")
    wid = sid * NC + cid
    nsl = pl.ds(sid * NPT, NPT)
    pltpu.sync_copy(zero_h.at[nsl, :], aggs.at[nsl, :])
    pltpu.sync_copy(col_h.at[pl.ds(wid * EPW, EPW)], col_v)
    plsc.subcore_barrier()

    def fire(j, b):
        ebase = wid * EPW + j * CE
        pltpu.make_async_copy(hw_h.at[col_v.at[pl.ds(j * CE, CE)]],
                              gath_b[b], gsem.at[b]).start()
        pltpu.make_async_copy(we_h.at[pl.ds(ebase, CE), :], we_b[b],
                              wsem.at[b]).start()
        pltpu.make_async_copy(row2_h.at[wid, j], row_b[b],
                              rsem.at[b]).start()

    def wait_data(j, b):
        pltpu.make_async_copy(hw_h.at[col_v.at[pl.ds(j * CE, CE)]],
                              gath_b[b], gsem.at[b]).wait()
        pltpu.make_async_copy(we_h.at[pl.ds(wid * EPW + j * CE, CE), :],
                              we_b[b], wsem.at[b]).wait()

    def wait_row(j, b):
        pltpu.make_async_copy(row2_h.at[wid, j], row_b[b],
                              rsem.at[b]).wait()

    def scat_start(b):
        pltpu.make_async_copy(gath_b[b], aggs.at[row_b[b].at[0]],
                              ssem.at[b]).start(add=True)

    def scat_wait(b):
        pltpu.make_async_copy(gath_b[b], aggs.at[row_b[b].at[0]],
                              ssem.at[b]).wait()

    def section(j, b, wait_prev):
        wait_data(j, b)
        gb = gath_b[b]
        wb = we_b[b]

        def rowmul(r, c2):
            for k in range(H // 16):
                sl = pl.ds(k * 16, 16)
                gb[r, sl] = gb[r, sl] * wb[r, sl]
            return c2

        lax.fori_loop(0, CE, rowmul, 0)
        wait_row(j, b)
        if wait_prev is not None:
            wait_prev()
        scat_start(b)

    for b in range(NBUF):
        fire(b, b)

    def outer(j2, c):
        for b in range(NBUF):
            j = j2 * NBUF + b
            bp = (b + NBUF - 1) % NBUF
            if b == 0:
                wait_prev = lambda: pl.when(j2 >= 1)(lambda: scat_wait(bp))
            else:
                wait_prev = lambda: scat_wait(bp)
            section(j, b, wait_prev)
            cond = (j2 >= 1) if b == 0 else (
                (j2 < NGROUP - 1) if b == NBUF - 1 else None)

            def refill():
                fire(j + NBUF - 1, bp)

            if cond is None:
                refill()
            else:
                pl.when(cond)(refill)
        return c

    lax.fori_loop(0, NGROUP, outer, 0)
    tb = (NCHUNK - 1) % NBUF
    section(NCHUNK - 1, tb, lambda: scat_wait((tb + NBUF - 1) % NBUF))
    scat_wait(tb)
    plsc.subcore_barrier()
    pltpu.sync_copy(aggs.at[nsl, :], out_h.at[cid, nsl, :])


def _sc_msg(hw, we, row2, col, zeros):
    mesh = plsc.VectorSubcoreMesh(core_axis_name="c", subcore_axis_name="s")
    fn = pl.kernel(
        _sc_msg_body,
        out_type=jax.ShapeDtypeStruct((NC, NP, H), F32),
        mesh=mesh,
        scratch_types=(
            [
                pltpu.VMEM((EPW,), jnp.int32),
                pltpu.VMEM_SHARED((NP, H), F32),
                pltpu.SemaphoreType.DMA((NBUF,)),
                pltpu.SemaphoreType.DMA((NBUF,)),
                pltpu.SemaphoreType.DMA((NBUF,)),
                pltpu.SemaphoreType.DMA((NBUF,)),
            ]
            + [pltpu.VMEM((CE, H), F32) for _ in range(2 * NBUF)]
            + [pltpu.VMEM((1, CE), jnp.int32) for _ in range(NBUF)]
        ),
    )
    return fn(hw, we, row2, col, zeros)


TN = 1000


def _tc_embed_body(z_ref, emb_ref, wl1_ref, h_ref, hw_ref):
    zt = z_ref[0, 0, :]
    oh = (zt[:, None] == lax.broadcasted_iota(jnp.int32, (TN, VOCAB), 1)
          ).astype(F32)
    h = jnp.dot(oh, emb_ref[...], preferred_element_type=F32)
    h_ref[...] = h
    hw_ref[...] = jnp.dot(h, wl1_ref[...], preferred_element_type=F32)


def _tc_embed(z, emb, wl10):
    z3 = z.reshape(N // TN, 1, TN)
    return pl.pallas_call(
        _tc_embed_body,
        grid=(N // TN,),
        in_specs=[
            pl.BlockSpec((1, 1, TN), lambda i: (i, 0, 0)),
            pl.BlockSpec((VOCAB, H), lambda i: (0, 0)),
            pl.BlockSpec((H, H), lambda i: (0, 0)),
        ],
        out_specs=[
            pl.BlockSpec((TN, H), lambda i: (i, 0)),
            pl.BlockSpec((TN, H), lambda i: (i, 0)),
        ],
        out_shape=[
            jax.ShapeDtypeStruct((N, H), F32),
            jax.ShapeDtypeStruct((NP, H), F32),
        ],
    )(z3, emb, wl10)


TE = 2000


def _tc_filter_body(dd_ref, eat_ref, wf1a_ref, wf1g_ref, bf1_ref,
                    wf2_ref, bf2_ref, o_ref):
    dd = dd_ref[0, 0, :]
    ew = jnp.sqrt(dd + 1e-12)
    step = CUT / (NG - 1)
    coeff = -0.5 / (step * step)
    offs = lax.broadcasted_iota(jnp.int32, (TE, NG), 1).astype(F32) * step
    diff = ew[:, None] - offs
    g = jnp.exp(coeff * diff * diff)
    C = 0.5 * (jnp.cos(ew * (jnp.pi / CUT)) + 1.0)
    ea = eat_ref[0]
    t = (jnp.dot(ea, wf1a_ref[...], preferred_element_type=F32)
         + jnp.dot(g, wf1g_ref[...], preferred_element_type=F32)
         + bf1_ref[...])
    t = ssp(t)
    o_ref[...] = (jnp.dot(t, wf2_ref[...], preferred_element_type=F32)
                  + bf2_ref[...]) * C[:, None]


def _tc_filter(dd3, ea3, wf1a, wf1g, bf1, wf2, bf2):
    return pl.pallas_call(
        _tc_filter_body,
        grid=(E // TE,),
        in_specs=[
            pl.BlockSpec((1, 1, TE), lambda i: (i, 0, 0)),
            pl.BlockSpec((1, TE, NB), lambda i: (i, 0, 0)),
            pl.BlockSpec((NB, H), lambda i: (0, 0)),
            pl.BlockSpec((NG, H), lambda i: (0, 0)),
            pl.BlockSpec((1, H), lambda i: (0, 0)),
            pl.BlockSpec((H, H), lambda i: (0, 0)),
            pl.BlockSpec((1, H), lambda i: (0, 0)),
        ],
        out_specs=pl.BlockSpec((TE, H), lambda i: (i, 0)),
        out_shape=jax.ShapeDtypeStruct((E, H), F32),
    )(dd3, ea3, wf1a, wf1g, bf1, wf2, bf2)


def _tc_update_body(agg_ref, h_ref, wl2_ref, bl2_ref, wl1n_ref,
                    hn_ref, hwn_ref):
    agg = agg_ref[0] + agg_ref[1]
    hn = h_ref[...] + jnp.dot(ssp(agg), wl2_ref[...],
                              preferred_element_type=F32) + bl2_ref[...]
    hn_ref[...] = hn
    hwn_ref[...] = jnp.dot(hn, wl1n_ref[...], preferred_element_type=F32)


def _tc_update(agg2, h, wl2, bl2, wl1n):
    return pl.pallas_call(
        _tc_update_body,
        grid=(N // TN,),
        in_specs=[
            pl.BlockSpec((NC, TN, H), lambda i: (0, i, 0)),
            pl.BlockSpec((TN, H), lambda i: (i, 0)),
            pl.BlockSpec((H, H), lambda i: (0, 0)),
            pl.BlockSpec((1, H), lambda i: (0, 0)),
            pl.BlockSpec((H, H), lambda i: (0, 0)),
        ],
        out_specs=[
            pl.BlockSpec((TN, H), lambda i: (i, 0)),
            pl.BlockSpec((TN, H), lambda i: (i, 0)),
        ],
        out_shape=[
            jax.ShapeDtypeStruct((N, H), F32),
            jax.ShapeDtypeStruct((NP, H), F32),
        ],
    )(agg2, h, wl2, bl2.reshape(1, H), wl1n)


def _tc_final_body(agg_ref, h_ref, wl2_ref, bl2_ref, wr1_ref, br1_ref,
                   wr2_ref, br2_ref, batch_ref, out_ref):
    agg = agg_ref[0] + agg_ref[1]
    h3 = h_ref[...] + jnp.dot(ssp(agg), wl2_ref[...],
                              preferred_element_type=F32) + bl2_ref[...]
    u = ssp(jnp.dot(h3, wr1_ref[...], preferred_element_type=F32)
            + br1_ref[...])
    hq = jnp.dot(u, wr2_ref[...], preferred_element_type=F32) + br2_ref[...]
    bt = batch_ref[0, 0, :]
    oh = (bt[:, None] == lax.broadcasted_iota(jnp.int32, (TN, NGRAPH), 1)
          ).astype(F32)
    contrib = lax.dot_general(oh, hq, (((0,), (0,)), ((), ())),
                              preferred_element_type=F32)

    @pl.when(pl.program_id(0) == 0)
    def _():
        out_ref[...] = jnp.zeros_like(out_ref)

    out_ref[...] += contrib


def _tc_final(agg2, h, wl2, bl2, Wr1, br1, Wr2, br2, batch):
    wr2p = jnp.pad(Wr2, ((0, 0), (0, H - Wr2.shape[1])))
    b3 = batch.reshape(N // TN, 1, TN)
    out128 = pl.pallas_call(
        _tc_final_body,
        grid=(N // TN,),
        in_specs=[
            pl.BlockSpec((NC, TN, H), lambda i: (0, i, 0)),
            pl.BlockSpec((TN, H), lambda i: (i, 0)),
            pl.BlockSpec((H, H), lambda i: (0, 0)),
            pl.BlockSpec((1, H), lambda i: (0, 0)),
            pl.BlockSpec((H, H // 2), lambda i: (0, 0)),
            pl.BlockSpec((1, H // 2), lambda i: (0, 0)),
            pl.BlockSpec((H // 2, H), lambda i: (0, 0)),
            pl.BlockSpec((1, 1), lambda i: (0, 0)),
            pl.BlockSpec((1, 1, TN), lambda i: (i, 0, 0)),
        ],
        out_specs=pl.BlockSpec((NGRAPH, H), lambda i: (0, 0)),
        out_shape=jax.ShapeDtypeStruct((NGRAPH, H), F32),
    )(agg2, h, wl2, bl2.reshape(1, H), Wr1, br1.reshape(1, H // 2),
      wr2p, br2.reshape(1, 1), b3)
    return out128[:, :1]


def kernel(z, pos, batch, edge_index, edge_attr, emb, Wf1, bf1, Wf2, bf2,
           Wl1, Wl2, bl2, Wr1, br1, Wr2, br2):
    row = edge_index[0].astype(jnp.int32)
    col = edge_index[1].astype(jnp.int32)
    px = pos[:, 0]
    py = pos[:, 1]
    pz = pos[:, 2]

    row2 = row.reshape(NW, NCHUNK, 1, CE)

    dd = _sc_dd(px, py, pz, row, col)
    dd3 = dd.reshape(E // TE, 1, TE)
    ea3 = edge_attr.reshape(E // TE, TE, NB)
    wf1a = Wf1[:, :NB, :]
    wf1g = Wf1[:, NB:, :]

    def filt(i):
        return _tc_filter(dd3, ea3, wf1a[i], wf1g[i], bf1[i].reshape(1, H),
                          Wf2[i], bf2[i].reshape(1, H))

    h, hw = _tc_embed(z.astype(jnp.int32), emb, Wl1[0])
    zeros = jnp.zeros((NP, H), F32)
    we_cur = filt(0)
    for i in range(NI - 1):
        agg2 = _sc_msg(hw, we_cur, row2, col, zeros)
        we_cur = filt(i + 1)
        h, hw = _tc_update(agg2, h, Wl2[i], bl2[i], Wl1[i + 1])
    agg2 = _sc_msg(hw, we_cur, row2, col, zeros)
    return _tc_final(agg2, h, Wl2[NI - 1], bl2[NI - 1],
                     Wr1, br1, Wr2, br2, batch.astype(jnp.int32))
